# baseline (speedup 1.0000x reference)
import functools

import jax
import jax.numpy as jnp
from jax.experimental import pallas as pl
from jax.experimental.pallas import tpu as pltpu


def _fused_gcn_kernel(x_ref, wt_ref, b_ref, adj_ref, o_ref):
    h = jnp.dot(adj_ref[...], x_ref[...], preferred_element_type=jnp.float32)
    y = jnp.dot(h, wt_ref[...], preferred_element_type=jnp.float32) + b_ref[...]
    o_ref[...] = jnp.maximum(y, 0.0)


def _pick_block_rows(n: int) -> int:
    best = 8
    for bm in range(8, 513, 8):
        if n % bm == 0:
            best = bm
    return best


@functools.partial(jax.jit, static_argnames=())
def _run(x, adj, wt, b2):
    n, d_in = x.shape
    d_out = wt.shape[1]
    bm = _pick_block_rows(n)
    grid = (n // bm,)
    return pl.pallas_call(
        _fused_gcn_kernel,
        grid=grid,
        in_specs=[
            pl.BlockSpec((n, d_in), lambda i: (0, 0)),
            pl.BlockSpec((d_in, d_out), lambda i: (0, 0)),
            pl.BlockSpec((1, d_out), lambda i: (0, 0)),
            pl.BlockSpec((bm, n), lambda i: (i, 0)),
        ],
        out_specs=pl.BlockSpec((bm, d_out), lambda i: (i, 0)),
        out_shape=jax.ShapeDtypeStruct((n, d_out), jnp.float32),
        compiler_params=pltpu.CompilerParams(
            dimension_semantics=("arbitrary",),
        ),
    )(x, wt, b2, adj)


def kernel(input, adj, W, b):
    wt = W.T
    b2 = b.reshape(1, -1)
    return _run(input, adj, wt, b2)
